# baseline (speedup 1.0000x reference)
import functools

import jax
import jax.numpy as jnp
import numpy as np
from jax import lax
from jax.experimental import pallas as pl
from jax.experimental.pallas import tpu as pltpu
from jax.experimental.pallas import tpu_sc as plsc

N = 10000
E = 320000
D = 128
H = 128
C = 64

NC = 2
NS = 16
NW = NC * NS
CHUNK = 80
CPW = E // (NW * CHUNK)
ACH = 128
CPWS = 160
NBUF = 4
RPS = 640
N_ACC = NS * RPS
E_PAD = NS * CPWS * ACH
CPW2 = E_PAD // (NW * ACH)
N_PAD_ROWS = N_ACC - N
ZROWS = 64
BR = 2000


_SC_PARAMS = pltpu.CompilerParams(use_tc_tiling_on_sc=False)


def _run_edge_pipeline(nch, gather, wait_gather, scatter, wait_scatter):
    half = NBUF // 2
    for k in range(NBUF):
        gather(k, k)
    for k in range(half):
        wait_gather(k, k)
        scatter(k, k)

    @pl.loop(NBUF, nch, step=NBUF)
    def _(i):
        for k in range(NBUF):
            bh = (k + half) % NBUF
            wait_gather(i + k - half, bh)
            scatter(i + k - half, bh)
            wait_scatter(i + k - NBUF, k)
            gather(i + k, k)

    for k in range(half):
        j = nch - half + k
        wait_gather(j, (half + k) % NBUF)
        scatter(j, (half + k) % NBUF)
    for k in range(NBUF):
        wait_scatter(nch - NBUF + k, k)


def _sc_degree(dst3d):
    mesh = plsc.VectorSubcoreMesh(core_axis_name="c", subcore_axis_name="s")

    @functools.partial(
        pl.kernel,
        out_type=jax.ShapeDtypeStruct((NC, N_ACC), jnp.float32),
        mesh=mesh,
        compiler_params=_SC_PARAMS,
        scratch_types=[
            pltpu.VMEM((CPW, CHUNK), jnp.int32),
            pltpu.VMEM((CHUNK,), jnp.float32),
            pltpu.VMEM((RPS,), jnp.float32),
            pltpu.VMEM_SHARED((N_ACC,), jnp.float32),
            pltpu.SemaphoreType.DMA,
        ],
    )
    def k(dst_hbm, out_hbm, idx_v, ones_v, zeros_v, acc_sh, sem):
        c = lax.axis_index("c")
        s = lax.axis_index("s")
        w = s * NC + c

        @pl.loop(0, CHUNK, step=16)
        def _(i):
            ones_v[pl.ds(i, 16)] = jnp.full((16,), 1.0, jnp.float32)

        @pl.loop(0, RPS, step=16)
        def _(i):
            zeros_v[pl.ds(i, 16)] = jnp.zeros((16,), jnp.float32)

        pltpu.sync_copy(zeros_v, acc_sh.at[pl.ds(s * RPS, RPS)])
        plsc.subcore_barrier()

        pltpu.sync_copy(dst_hbm.at[w], idx_v)

        @pl.loop(0, CPW, step=5)
        def _(i0):
            for j in range(5):
                pltpu.async_copy(ones_v, acc_sh.at[idx_v.at[i0 + j]], sem,
                                 add=True)
            for j in range(5):
                pltpu.make_async_copy(ones_v, acc_sh.at[idx_v.at[i0 + j]],
                                      sem).wait()

        plsc.subcore_barrier()
        pltpu.sync_copy(acc_sh.at[pl.ds(s * RPS, RPS)],
                        out_hbm.at[c, pl.ds(s * RPS, RPS)])

    return k(dst3d)


def _sc_edge_agg(yr, src3d, dst3d):
    Wc = yr.shape[1]
    M = 128 // Wc
    mesh = plsc.VectorSubcoreMesh(core_axis_name="c", subcore_axis_name="s")

    @functools.partial(
        pl.kernel,
        out_type=jax.ShapeDtypeStruct((N_ACC, 128), jnp.float32),
        mesh=mesh,
        compiler_params=_SC_PARAMS,
        scratch_types=[
            pltpu.VMEM((CPWS, ACH), jnp.int32),
            pltpu.VMEM((CPWS, ACH), jnp.int32),
            [pltpu.VMEM((ACH, Wc), jnp.float32) for _ in range(NBUF)],
            pltpu.VMEM((ZROWS, Wc), jnp.float32),
            pltpu.VMEM_SHARED((N_ACC, Wc), jnp.float32),
            [pltpu.SemaphoreType.DMA for _ in range(NBUF)],
            [pltpu.SemaphoreType.DMA for _ in range(NBUF)],
        ],
    )
    def k(y_hbm, src_hbm, dst_hbm, out_hbm, src_v, dst_v, bufs,
          zeros_v, acc_sh, gsems, ssems):
        c = lax.axis_index("c")
        s = lax.axis_index("s")

        @pl.loop(0, ZROWS)
        def _(i):
            @pl.loop(0, Wc, step=16)
            def _(j):
                zeros_v[i, pl.ds(j, 16)] = jnp.zeros((16,), jnp.float32)

        @pl.loop(0, RPS, step=ZROWS)
        def _(r):
            pltpu.sync_copy(zeros_v, acc_sh.at[pl.ds(s * RPS + r, ZROWS)])

        plsc.subcore_barrier()

        pltpu.sync_copy(src_hbm.at[s], src_v)
        pltpu.sync_copy(dst_hbm.at[s], dst_v)

        @pl.loop(0, CPWS)
        def _(i):
            @pl.loop(0, ACH, step=16)
            def _(j):
                v = src_v[i, pl.ds(j, 16)]
                src_v[i, pl.ds(j, 16)] = v * M + c

        def gather(i, b):
            pltpu.async_copy(y_hbm.at[src_v.at[i]], bufs[b], gsems[b])

        def wait_gather(i, b):
            pltpu.make_async_copy(y_hbm.at[src_v.at[i]], bufs[b],
                                  gsems[b]).wait()

        def scatter(i, b):
            pltpu.async_copy(bufs[b], acc_sh.at[dst_v.at[i]], ssems[b],
                             add=True)

        def wait_scatter(i, b):
            pltpu.make_async_copy(bufs[b], acc_sh.at[dst_v.at[i]],
                                  ssems[b]).wait()

        _run_edge_pipeline(CPWS, gather, wait_gather, scatter, wait_scatter)

        plsc.subcore_barrier()
        pltpu.sync_copy(acc_sh.at[pl.ds(s * RPS, RPS)],
                        out_hbm.at[pl.ds(s * RPS, RPS),
                                   pl.ds(c * Wc, Wc)])

    return k(yr, src3d, dst3d)


def _sc_edge_agg_es(yr, src3d, dst3d):
    Wc = yr.shape[1]
    mesh = plsc.VectorSubcoreMesh(core_axis_name="c", subcore_axis_name="s")

    @functools.partial(
        pl.kernel,
        out_type=jax.ShapeDtypeStruct((N_ACC, 128), jnp.float32),
        mesh=mesh,
        compiler_params=_SC_PARAMS,
        scratch_types=[
            pltpu.VMEM((CPW2, ACH), jnp.int32),
            pltpu.VMEM((CPW2, ACH), jnp.int32),
            [pltpu.VMEM((ACH, Wc), jnp.float32) for _ in range(NBUF)],
            pltpu.VMEM((ZROWS, Wc), jnp.float32),
            pltpu.VMEM_SHARED((N_ACC, Wc), jnp.float32),
            [pltpu.SemaphoreType.DMA for _ in range(NBUF)],
            [pltpu.SemaphoreType.DMA for _ in range(NBUF)],
        ],
    )
    def k(y_hbm, src_hbm, dst_hbm, out_hbm, src_v, dst_v, bufs,
          zeros_v, acc_sh, gsems, ssems):
        c = lax.axis_index("c")
        s = lax.axis_index("s")
        w = s * NC + c

        @pl.loop(0, ZROWS)
        def _(i):
            @pl.loop(0, Wc, step=16)
            def _(j):
                zeros_v[i, pl.ds(j, 16)] = jnp.zeros((16,), jnp.float32)

        @pl.loop(0, RPS, step=ZROWS)
        def _(r):
            pltpu.sync_copy(zeros_v, acc_sh.at[pl.ds(s * RPS + r, ZROWS)])

        plsc.subcore_barrier()

        pltpu.sync_copy(src_hbm.at[w], src_v)
        pltpu.sync_copy(dst_hbm.at[w], dst_v)

        @pl.loop(0, CPW2)
        def _(i):
            @pl.loop(0, ACH, step=16)
            def _(j):
                v = src_v[i, pl.ds(j, 16)]
                src_v[i, pl.ds(j, 16)] = v * 2

        def gather(i, b):
            pltpu.async_copy(y_hbm.at[src_v.at[i]], bufs[b], gsems[b])

        def wait_gather(i, b):
            pltpu.make_async_copy(y_hbm.at[src_v.at[i]], bufs[b],
                                  gsems[b]).wait()

        def scatter(i, b):
            pltpu.async_copy(bufs[b], acc_sh.at[dst_v.at[i]], ssems[b],
                             add=True)

        def wait_scatter(i, b):
            pltpu.make_async_copy(bufs[b], acc_sh.at[dst_v.at[i]],
                                  ssems[b]).wait()

        _run_edge_pipeline(CPW2, gather, wait_gather, scatter, wait_scatter)

        plsc.subcore_barrier()
        pltpu.sync_copy(acc_sh.at[pl.ds(s * RPS, RPS)],
                        out_hbm.at[pl.ds(s * RPS, RPS),
                                   pl.ds(c * Wc, Wc)])

    return k(yr, src3d, dst3d)


def _tc_layer1(x, W1, degp0, degp1):
    def body(x_ref, w_ref, d0_ref, d1_ref, y_ref, dinv_ref):
        deg = d0_ref[...] + d1_ref[...] + 1.0
        dinv = lax.rsqrt(deg)
        xw = jnp.dot(x_ref[...], w_ref[...],
                     preferred_element_type=jnp.float32)
        y_ref[...] = xw * dinv
        dinv_ref[...] = jnp.broadcast_to(dinv, (BR, H))

    return pl.pallas_call(
        body,
        grid=(N // BR,),
        in_specs=[
            pl.BlockSpec((BR, D), lambda i: (i, 0)),
            pl.BlockSpec((D, H), lambda i: (0, 0)),
            pl.BlockSpec((BR, 1), lambda i: (i, 0)),
            pl.BlockSpec((BR, 1), lambda i: (i, 0)),
        ],
        out_specs=[
            pl.BlockSpec((BR, H), lambda i: (i, 0)),
            pl.BlockSpec((BR, H), lambda i: (i, 0)),
        ],
        out_shape=[
            jax.ShapeDtypeStruct((N, H), jnp.float32),
            jax.ShapeDtypeStruct((N, H), jnp.float32),
        ],
    )(x, W1, degp0, degp1)


def _tc_layer2(y1, aggp, dinvb, b1r, W2):
    def body(y_ref, p_ref, dinv_ref, b_ref, w_ref, y2_ref):
        dinv = dinv_ref[...]
        agg = p_ref[...] + y_ref[...]
        h = jnp.maximum(dinv * agg + b_ref[...], 0.0)
        hw = jnp.dot(h, w_ref[...], preferred_element_type=jnp.float32)
        y2 = hw * dinv[:, :C]
        y2_ref[...] = jnp.concatenate([y2, y2], axis=1)

    return pl.pallas_call(
        body,
        grid=(N // BR,),
        in_specs=[
            pl.BlockSpec((BR, H), lambda i: (i, 0)),
            pl.BlockSpec((BR, H), lambda i: (i, 0)),
            pl.BlockSpec((BR, H), lambda i: (i, 0)),
            pl.BlockSpec((1, H), lambda i: (0, 0)),
            pl.BlockSpec((H, C), lambda i: (0, 0)),
        ],
        out_specs=pl.BlockSpec((BR, H), lambda i: (i, 0)),
        out_shape=jax.ShapeDtypeStruct((N, H), jnp.float32),
    )(y1, aggp, dinvb, b1r, W2)


def _tc_layer3(y2w, aggp, dinvb, b2r):
    def body(y_ref, p_ref, dinv_ref, b_ref, o_ref):
        agg = p_ref[...][:, :C] + y_ref[...][:, :C]
        o = dinv_ref[...][:, :C] * agg + b_ref[...]
        m = jnp.max(o, axis=1, keepdims=True)
        lse = jnp.log(jnp.sum(jnp.exp(o - m), axis=1, keepdims=True)) + m
        o_ref[...] = o - lse

    return pl.pallas_call(
        body,
        grid=(N // BR,),
        in_specs=[
            pl.BlockSpec((BR, H), lambda i: (i, 0)),
            pl.BlockSpec((BR, H), lambda i: (i, 0)),
            pl.BlockSpec((BR, H), lambda i: (i, 0)),
            pl.BlockSpec((1, C), lambda i: (0, 0)),
        ],
        out_specs=pl.BlockSpec((BR, C), lambda i: (i, 0)),
        out_shape=jax.ShapeDtypeStruct((N, C), jnp.float32),
    )(y2w, aggp, dinvb, b2r)


def kernel(x, edge_index, W1, b1, W2, b2):
    pad_n = E_PAD - E
    pad_iota = np.arange(pad_n, dtype=np.int32)
    dst_deg3d = edge_index[1].reshape(NW, CPW, CHUNK)
    src3d = jnp.concatenate(
        [edge_index[0], jnp.asarray(pad_iota % N)]).reshape(NS, CPWS, ACH)
    dst3d = jnp.concatenate(
        [edge_index[1],
         jnp.asarray(N + pad_iota % N_PAD_ROWS)]).reshape(NS, CPWS, ACH)

    degp = _sc_degree(dst_deg3d)
    degp0 = degp[0, :N].reshape(N, 1)
    degp1 = degp[1, :N].reshape(N, 1)

    y1, dinvb = _tc_layer1(x, W1, degp0, degp1)
    aggp1 = _sc_edge_agg(y1.reshape(2 * N, H // 2), src3d, dst3d)
    y2w = _tc_layer2(y1, aggp1, dinvb, b1.reshape(1, H), W2)
    aggp2 = _sc_edge_agg(y2w.reshape(4 * N, H // 4), src3d, dst3d)
    return _tc_layer3(y2w, aggp2, dinvb, b2.reshape(1, C))


# speedup vs baseline: 1.1028x; 1.1028x over previous
import functools

import jax
import jax.numpy as jnp
import numpy as np
from jax import lax
from jax.experimental import pallas as pl
from jax.experimental.pallas import tpu as pltpu
from jax.experimental.pallas import tpu_sc as plsc

N = 10000
E = 320000
D = 128
H = 128
C = 64

NC = 2
NS = 16
NW = NC * NS
CHUNK = 80
CPW = E // (NW * CHUNK)
ACH = 128
CPWS = 160
NBUF = 4
RPS = 640
N_ACC = NS * RPS
E_PAD = NS * CPWS * ACH
CPW2 = E_PAD // (NW * ACH)
N_PAD_ROWS = N_ACC - N
ZROWS = 64
BR = 2000


_SC_PARAMS = pltpu.CompilerParams(use_tc_tiling_on_sc=False)


def _run_edge_pipeline(nch, gather, wait_gather, scatter, wait_scatter):
    for b in range(NBUF):
        gather(b, b)

    @pl.loop(0, nch - NBUF, step=NBUF)
    def _(i):
        for b in range(NBUF):
            wait_gather(i + b, b)
            scatter(i + b, b)
        for b in range(NBUF):
            wait_scatter(i + b, b)
            gather(i + NBUF + b, b)

    for b in range(NBUF):
        wait_gather(nch - NBUF + b, b)
        scatter(nch - NBUF + b, b)
    for b in range(NBUF):
        wait_scatter(nch - NBUF + b, b)


def _sc_degree(dst3d):
    mesh = plsc.VectorSubcoreMesh(core_axis_name="c", subcore_axis_name="s")

    @functools.partial(
        pl.kernel,
        out_type=jax.ShapeDtypeStruct((NC, N_ACC), jnp.float32),
        mesh=mesh,
        compiler_params=_SC_PARAMS,
        scratch_types=[
            pltpu.VMEM((CPW, CHUNK), jnp.int32),
            pltpu.VMEM((CHUNK,), jnp.float32),
            pltpu.VMEM((RPS,), jnp.float32),
            pltpu.VMEM_SHARED((N_ACC,), jnp.float32),
            pltpu.SemaphoreType.DMA,
        ],
    )
    def k(dst_hbm, out_hbm, idx_v, ones_v, zeros_v, acc_sh, sem):
        c = lax.axis_index("c")
        s = lax.axis_index("s")
        w = s * NC + c

        @pl.loop(0, CHUNK, step=16)
        def _(i):
            ones_v[pl.ds(i, 16)] = jnp.full((16,), 1.0, jnp.float32)

        @pl.loop(0, RPS, step=16)
        def _(i):
            zeros_v[pl.ds(i, 16)] = jnp.zeros((16,), jnp.float32)

        pltpu.sync_copy(zeros_v, acc_sh.at[pl.ds(s * RPS, RPS)])
        plsc.subcore_barrier()

        pltpu.sync_copy(dst_hbm.at[w], idx_v)

        @pl.loop(0, CPW, step=5)
        def _(i0):
            for j in range(5):
                pltpu.async_copy(ones_v, acc_sh.at[idx_v.at[i0 + j]], sem,
                                 add=True)
            for j in range(5):
                pltpu.make_async_copy(ones_v, acc_sh.at[idx_v.at[i0 + j]],
                                      sem).wait()

        plsc.subcore_barrier()
        pltpu.sync_copy(acc_sh.at[pl.ds(s * RPS, RPS)],
                        out_hbm.at[c, pl.ds(s * RPS, RPS)])

    return k(dst3d)


def _sc_edge_agg(yr, src3d, dst3d):
    Wc = yr.shape[1]
    M = 128 // Wc
    mesh = plsc.VectorSubcoreMesh(core_axis_name="c", subcore_axis_name="s")

    @functools.partial(
        pl.kernel,
        out_type=jax.ShapeDtypeStruct((N_ACC, 128), jnp.float32),
        mesh=mesh,
        compiler_params=_SC_PARAMS,
        scratch_types=[
            pltpu.VMEM((CPWS, ACH), jnp.int32),
            pltpu.VMEM((CPWS, ACH), jnp.int32),
            [pltpu.VMEM((ACH, Wc), jnp.float32) for _ in range(NBUF)],
            pltpu.VMEM((ZROWS, Wc), jnp.float32),
            pltpu.VMEM_SHARED((N_ACC, Wc), jnp.float32),
            [pltpu.SemaphoreType.DMA for _ in range(NBUF)],
            [pltpu.SemaphoreType.DMA for _ in range(NBUF)],
        ],
    )
    def k(y_hbm, src_hbm, dst_hbm, out_hbm, src_v, dst_v, bufs,
          zeros_v, acc_sh, gsems, ssems):
        c = lax.axis_index("c")
        s = lax.axis_index("s")

        @pl.loop(0, ZROWS)
        def _(i):
            @pl.loop(0, Wc, step=16)
            def _(j):
                zeros_v[i, pl.ds(j, 16)] = jnp.zeros((16,), jnp.float32)

        @pl.loop(0, RPS, step=ZROWS)
        def _(r):
            pltpu.sync_copy(zeros_v, acc_sh.at[pl.ds(s * RPS + r, ZROWS)])

        plsc.subcore_barrier()

        pltpu.sync_copy(src_hbm.at[s], src_v)
        pltpu.sync_copy(dst_hbm.at[s], dst_v)

        @pl.loop(0, CPWS)
        def _(i):
            @pl.loop(0, ACH, step=16)
            def _(j):
                v = src_v[i, pl.ds(j, 16)]
                src_v[i, pl.ds(j, 16)] = v * M + c

        def gather(i, b):
            pltpu.async_copy(y_hbm.at[src_v.at[i]], bufs[b], gsems[b])

        def wait_gather(i, b):
            pltpu.make_async_copy(y_hbm.at[src_v.at[i]], bufs[b],
                                  gsems[b]).wait()

        def scatter(i, b):
            pltpu.async_copy(bufs[b], acc_sh.at[dst_v.at[i]], ssems[b],
                             add=True)

        def wait_scatter(i, b):
            pltpu.make_async_copy(bufs[b], acc_sh.at[dst_v.at[i]],
                                  ssems[b]).wait()

        _run_edge_pipeline(CPWS, gather, wait_gather, scatter, wait_scatter)

        plsc.subcore_barrier()
        pltpu.sync_copy(acc_sh.at[pl.ds(s * RPS, RPS)],
                        out_hbm.at[pl.ds(s * RPS, RPS),
                                   pl.ds(c * Wc, Wc)])

    return k(yr, src3d, dst3d)


def _sc_edge_agg_es(yr, src3d, dst3d):
    Wc = yr.shape[1]
    mesh = plsc.VectorSubcoreMesh(core_axis_name="c", subcore_axis_name="s")

    @functools.partial(
        pl.kernel,
        out_type=jax.ShapeDtypeStruct((N_ACC, 128), jnp.float32),
        mesh=mesh,
        compiler_params=_SC_PARAMS,
        scratch_types=[
            pltpu.VMEM((CPW2, ACH), jnp.int32),
            pltpu.VMEM((CPW2, ACH), jnp.int32),
            [pltpu.VMEM((ACH, Wc), jnp.float32) for _ in range(NBUF)],
            pltpu.VMEM((ZROWS, Wc), jnp.float32),
            pltpu.VMEM_SHARED((N_ACC, Wc), jnp.float32),
            [pltpu.SemaphoreType.DMA for _ in range(NBUF)],
            [pltpu.SemaphoreType.DMA for _ in range(NBUF)],
        ],
    )
    def k(y_hbm, src_hbm, dst_hbm, out_hbm, src_v, dst_v, bufs,
          zeros_v, acc_sh, gsems, ssems):
        c = lax.axis_index("c")
        s = lax.axis_index("s")
        w = s * NC + c

        @pl.loop(0, ZROWS)
        def _(i):
            @pl.loop(0, Wc, step=16)
            def _(j):
                zeros_v[i, pl.ds(j, 16)] = jnp.zeros((16,), jnp.float32)

        @pl.loop(0, RPS, step=ZROWS)
        def _(r):
            pltpu.sync_copy(zeros_v, acc_sh.at[pl.ds(s * RPS + r, ZROWS)])

        plsc.subcore_barrier()

        pltpu.sync_copy(src_hbm.at[w], src_v)
        pltpu.sync_copy(dst_hbm.at[w], dst_v)

        @pl.loop(0, CPW2)
        def _(i):
            @pl.loop(0, ACH, step=16)
            def _(j):
                v = src_v[i, pl.ds(j, 16)]
                src_v[i, pl.ds(j, 16)] = v * 2

        def gather(i, b):
            pltpu.async_copy(y_hbm.at[src_v.at[i]], bufs[b], gsems[b])

        def wait_gather(i, b):
            pltpu.make_async_copy(y_hbm.at[src_v.at[i]], bufs[b],
                                  gsems[b]).wait()

        def scatter(i, b):
            pltpu.async_copy(bufs[b], acc_sh.at[dst_v.at[i]], ssems[b],
                             add=True)

        def wait_scatter(i, b):
            pltpu.make_async_copy(bufs[b], acc_sh.at[dst_v.at[i]],
                                  ssems[b]).wait()

        _run_edge_pipeline(CPW2, gather, wait_gather, scatter, wait_scatter)

        plsc.subcore_barrier()
        pltpu.sync_copy(acc_sh.at[pl.ds(s * RPS, RPS)],
                        out_hbm.at[pl.ds(s * RPS, RPS),
                                   pl.ds(c * Wc, Wc)])

    return k(yr, src3d, dst3d)


def _tc_layer1(x, W1, degp0, degp1):
    def body(x_ref, w_ref, d0_ref, d1_ref, y_ref, dinv_ref):
        deg = d0_ref[...] + d1_ref[...] + 1.0
        dinv = lax.rsqrt(deg)
        xw = jnp.dot(x_ref[...], w_ref[...],
                     preferred_element_type=jnp.float32)
        y_ref[...] = xw * dinv
        dinv_ref[...] = jnp.broadcast_to(dinv, (BR, H))

    return pl.pallas_call(
        body,
        grid=(N // BR,),
        in_specs=[
            pl.BlockSpec((BR, D), lambda i: (i, 0)),
            pl.BlockSpec((D, H), lambda i: (0, 0)),
            pl.BlockSpec((BR, 1), lambda i: (i, 0)),
            pl.BlockSpec((BR, 1), lambda i: (i, 0)),
        ],
        out_specs=[
            pl.BlockSpec((BR, H), lambda i: (i, 0)),
            pl.BlockSpec((BR, H), lambda i: (i, 0)),
        ],
        out_shape=[
            jax.ShapeDtypeStruct((N, H), jnp.float32),
            jax.ShapeDtypeStruct((N, H), jnp.float32),
        ],
    )(x, W1, degp0, degp1)


def _tc_layer2(y1, aggp, dinvb, b1r, W2):
    def body(y_ref, p_ref, dinv_ref, b_ref, w_ref, y2_ref):
        dinv = dinv_ref[...]
        agg = p_ref[...] + y_ref[...]
        h = jnp.maximum(dinv * agg + b_ref[...], 0.0)
        hw = jnp.dot(h, w_ref[...], preferred_element_type=jnp.float32)
        y2 = hw * dinv[:, :C]
        y2_ref[...] = jnp.concatenate([y2, y2], axis=1)

    return pl.pallas_call(
        body,
        grid=(N // BR,),
        in_specs=[
            pl.BlockSpec((BR, H), lambda i: (i, 0)),
            pl.BlockSpec((BR, H), lambda i: (i, 0)),
            pl.BlockSpec((BR, H), lambda i: (i, 0)),
            pl.BlockSpec((1, H), lambda i: (0, 0)),
            pl.BlockSpec((H, C), lambda i: (0, 0)),
        ],
        out_specs=pl.BlockSpec((BR, H), lambda i: (i, 0)),
        out_shape=jax.ShapeDtypeStruct((N, H), jnp.float32),
    )(y1, aggp, dinvb, b1r, W2)


def _tc_layer3(y2w, aggp, dinvb, b2r):
    def body(y_ref, p_ref, dinv_ref, b_ref, o_ref):
        p = p_ref[...]
        agg = p[:, :C] + p[:, C:] + y_ref[...][:, :C]
        o = dinv_ref[...][:, :C] * agg + b_ref[...]
        m = jnp.max(o, axis=1, keepdims=True)
        lse = jnp.log(jnp.sum(jnp.exp(o - m), axis=1, keepdims=True)) + m
        o_ref[...] = o - lse

    return pl.pallas_call(
        body,
        grid=(N // BR,),
        in_specs=[
            pl.BlockSpec((BR, H), lambda i: (i, 0)),
            pl.BlockSpec((BR, H), lambda i: (i, 0)),
            pl.BlockSpec((BR, H), lambda i: (i, 0)),
            pl.BlockSpec((1, C), lambda i: (0, 0)),
        ],
        out_specs=pl.BlockSpec((BR, C), lambda i: (i, 0)),
        out_shape=jax.ShapeDtypeStruct((N, C), jnp.float32),
    )(y2w, aggp, dinvb, b2r)


def kernel(x, edge_index, W1, b1, W2, b2):
    pad_n = E_PAD - E
    pad_iota = np.arange(pad_n, dtype=np.int32)
    dst_deg3d = edge_index[1].reshape(NW, CPW, CHUNK)
    src3d = jnp.concatenate(
        [edge_index[0], jnp.asarray(pad_iota % N)]).reshape(NS, CPWS, ACH)
    dst3d = jnp.concatenate(
        [edge_index[1],
         jnp.asarray(N + pad_iota % N_PAD_ROWS)]).reshape(NS, CPWS, ACH)

    degp = _sc_degree(dst_deg3d)
    degp0 = degp[0, :N].reshape(N, 1)
    degp1 = degp[1, :N].reshape(N, 1)

    y1, dinvb = _tc_layer1(x, W1, degp0, degp1)
    aggp1 = _sc_edge_agg(y1.reshape(2 * N, H // 2), src3d, dst3d)
    y2w = _tc_layer2(y1, aggp1, dinvb, b1.reshape(1, H), W2)
    aggp2 = _sc_edge_agg_es(y2w.reshape(2 * N, C),
                            src3d.reshape(NW, CPW2, ACH),
                            dst3d.reshape(NW, CPW2, ACH))
    return _tc_layer3(y2w, aggp2, dinvb, b2.reshape(1, C))


# speedup vs baseline: 1.1191x; 1.0147x over previous
import functools

import jax
import jax.numpy as jnp
import numpy as np
from jax import lax
from jax.experimental import pallas as pl
from jax.experimental.pallas import tpu as pltpu
from jax.experimental.pallas import tpu_sc as plsc

N = 10000
E = 320000
D = 128
H = 128
C = 64

NC = 2
NS = 16
NW = NC * NS
CHUNK = 80
CPW = E // (NW * CHUNK)
ACH = 128
CPWS = 160
NBUF = 5
RPS = 640
N_ACC = NS * RPS
E_PAD = NS * CPWS * ACH
CPW2 = E_PAD // (NW * ACH)
N_PAD_ROWS = N_ACC - N
ZROWS = 64
BR = 2000


_SC_PARAMS = pltpu.CompilerParams(use_tc_tiling_on_sc=False)


def _run_edge_pipeline(nch, gather, wait_gather, scatter, wait_scatter):
    for b in range(NBUF):
        gather(b, b)

    @pl.loop(0, nch - NBUF, step=NBUF)
    def _(i):
        for b in range(NBUF):
            wait_gather(i + b, b)
            scatter(i + b, b)
        for b in range(NBUF):
            wait_scatter(i + b, b)
            gather(i + NBUF + b, b)

    for b in range(NBUF):
        wait_gather(nch - NBUF + b, b)
        scatter(nch - NBUF + b, b)
    for b in range(NBUF):
        wait_scatter(nch - NBUF + b, b)


def _sc_degree(dst3d):
    mesh = plsc.VectorSubcoreMesh(core_axis_name="c", subcore_axis_name="s")

    @functools.partial(
        pl.kernel,
        out_type=jax.ShapeDtypeStruct((NC, N_ACC), jnp.float32),
        mesh=mesh,
        compiler_params=_SC_PARAMS,
        scratch_types=[
            pltpu.VMEM((CPW, CHUNK), jnp.int32),
            pltpu.VMEM((CHUNK,), jnp.float32),
            pltpu.VMEM((RPS,), jnp.float32),
            pltpu.VMEM_SHARED((N_ACC,), jnp.float32),
            pltpu.SemaphoreType.DMA,
        ],
    )
    def k(dst_hbm, out_hbm, idx_v, ones_v, zeros_v, acc_sh, sem):
        c = lax.axis_index("c")
        s = lax.axis_index("s")
        w = s * NC + c

        @pl.loop(0, CHUNK, step=16)
        def _(i):
            ones_v[pl.ds(i, 16)] = jnp.full((16,), 1.0, jnp.float32)

        @pl.loop(0, RPS, step=16)
        def _(i):
            zeros_v[pl.ds(i, 16)] = jnp.zeros((16,), jnp.float32)

        pltpu.sync_copy(zeros_v, acc_sh.at[pl.ds(s * RPS, RPS)])
        plsc.subcore_barrier()

        pltpu.sync_copy(dst_hbm.at[w], idx_v)

        @pl.loop(0, CPW, step=5)
        def _(i0):
            for j in range(5):
                pltpu.async_copy(ones_v, acc_sh.at[idx_v.at[i0 + j]], sem,
                                 add=True)
            for j in range(5):
                pltpu.make_async_copy(ones_v, acc_sh.at[idx_v.at[i0 + j]],
                                      sem).wait()

        plsc.subcore_barrier()
        pltpu.sync_copy(acc_sh.at[pl.ds(s * RPS, RPS)],
                        out_hbm.at[c, pl.ds(s * RPS, RPS)])

    return k(dst3d)


def _sc_edge_agg(yr, src3d, dst3d):
    Wc = yr.shape[1]
    M = 128 // Wc
    mesh = plsc.VectorSubcoreMesh(core_axis_name="c", subcore_axis_name="s")

    @functools.partial(
        pl.kernel,
        out_type=jax.ShapeDtypeStruct((N_ACC, 128), jnp.float32),
        mesh=mesh,
        compiler_params=_SC_PARAMS,
        scratch_types=[
            pltpu.VMEM((CPWS, ACH), jnp.int32),
            pltpu.VMEM((CPWS, ACH), jnp.int32),
            [pltpu.VMEM((ACH, Wc), jnp.float32) for _ in range(NBUF)],
            pltpu.VMEM((ZROWS, Wc), jnp.float32),
            pltpu.VMEM_SHARED((N_ACC, Wc), jnp.float32),
            [pltpu.SemaphoreType.DMA for _ in range(NBUF)],
            [pltpu.SemaphoreType.DMA for _ in range(NBUF)],
        ],
    )
    def k(y_hbm, src_hbm, dst_hbm, out_hbm, src_v, dst_v, bufs,
          zeros_v, acc_sh, gsems, ssems):
        c = lax.axis_index("c")
        s = lax.axis_index("s")

        @pl.loop(0, ZROWS)
        def _(i):
            @pl.loop(0, Wc, step=16)
            def _(j):
                zeros_v[i, pl.ds(j, 16)] = jnp.zeros((16,), jnp.float32)

        @pl.loop(0, RPS, step=ZROWS)
        def _(r):
            pltpu.sync_copy(zeros_v, acc_sh.at[pl.ds(s * RPS + r, ZROWS)])

        plsc.subcore_barrier()

        pltpu.sync_copy(src_hbm.at[s], src_v)
        pltpu.sync_copy(dst_hbm.at[s], dst_v)

        @pl.loop(0, CPWS)
        def _(i):
            @pl.loop(0, ACH, step=16)
            def _(j):
                v = src_v[i, pl.ds(j, 16)]
                src_v[i, pl.ds(j, 16)] = v * M + c

        def gather(i, b):
            pltpu.async_copy(y_hbm.at[src_v.at[i]], bufs[b], gsems[b])

        def wait_gather(i, b):
            pltpu.make_async_copy(y_hbm.at[src_v.at[i]], bufs[b],
                                  gsems[b]).wait()

        def scatter(i, b):
            pltpu.async_copy(bufs[b], acc_sh.at[dst_v.at[i]], ssems[b],
                             add=True)

        def wait_scatter(i, b):
            pltpu.make_async_copy(bufs[b], acc_sh.at[dst_v.at[i]],
                                  ssems[b]).wait()

        _run_edge_pipeline(CPWS, gather, wait_gather, scatter, wait_scatter)

        plsc.subcore_barrier()
        pltpu.sync_copy(acc_sh.at[pl.ds(s * RPS, RPS)],
                        out_hbm.at[pl.ds(s * RPS, RPS),
                                   pl.ds(c * Wc, Wc)])

    return k(yr, src3d, dst3d)


def _sc_edge_agg_es(yr, src3d, dst3d):
    Wc = yr.shape[1]
    mesh = plsc.VectorSubcoreMesh(core_axis_name="c", subcore_axis_name="s")

    @functools.partial(
        pl.kernel,
        out_type=jax.ShapeDtypeStruct((N_ACC, 128), jnp.float32),
        mesh=mesh,
        compiler_params=_SC_PARAMS,
        scratch_types=[
            pltpu.VMEM((CPW2, ACH), jnp.int32),
            pltpu.VMEM((CPW2, ACH), jnp.int32),
            [pltpu.VMEM((ACH, Wc), jnp.float32) for _ in range(NBUF)],
            pltpu.VMEM((ZROWS, Wc), jnp.float32),
            pltpu.VMEM_SHARED((N_ACC, Wc), jnp.float32),
            [pltpu.SemaphoreType.DMA for _ in range(NBUF)],
            [pltpu.SemaphoreType.DMA for _ in range(NBUF)],
        ],
    )
    def k(y_hbm, src_hbm, dst_hbm, out_hbm, src_v, dst_v, bufs,
          zeros_v, acc_sh, gsems, ssems):
        c = lax.axis_index("c")
        s = lax.axis_index("s")
        w = s * NC + c

        @pl.loop(0, ZROWS)
        def _(i):
            @pl.loop(0, Wc, step=16)
            def _(j):
                zeros_v[i, pl.ds(j, 16)] = jnp.zeros((16,), jnp.float32)

        @pl.loop(0, RPS, step=ZROWS)
        def _(r):
            pltpu.sync_copy(zeros_v, acc_sh.at[pl.ds(s * RPS + r, ZROWS)])

        plsc.subcore_barrier()

        pltpu.sync_copy(src_hbm.at[w], src_v)
        pltpu.sync_copy(dst_hbm.at[w], dst_v)

        @pl.loop(0, CPW2)
        def _(i):
            @pl.loop(0, ACH, step=16)
            def _(j):
                v = src_v[i, pl.ds(j, 16)]
                src_v[i, pl.ds(j, 16)] = v * 2

        def gather(i, b):
            pltpu.async_copy(y_hbm.at[src_v.at[i]], bufs[b], gsems[b])

        def wait_gather(i, b):
            pltpu.make_async_copy(y_hbm.at[src_v.at[i]], bufs[b],
                                  gsems[b]).wait()

        def scatter(i, b):
            pltpu.async_copy(bufs[b], acc_sh.at[dst_v.at[i]], ssems[b],
                             add=True)

        def wait_scatter(i, b):
            pltpu.make_async_copy(bufs[b], acc_sh.at[dst_v.at[i]],
                                  ssems[b]).wait()

        _run_edge_pipeline(CPW2, gather, wait_gather, scatter, wait_scatter)

        plsc.subcore_barrier()
        pltpu.sync_copy(acc_sh.at[pl.ds(s * RPS, RPS)],
                        out_hbm.at[pl.ds(s * RPS, RPS),
                                   pl.ds(c * Wc, Wc)])

    return k(yr, src3d, dst3d)


def _tc_layer1(x, W1, degp0, degp1):
    def body(x_ref, w_ref, d0_ref, d1_ref, y_ref, dinv_ref):
        deg = d0_ref[...] + d1_ref[...] + 1.0
        dinv = lax.rsqrt(deg)
        xw = jnp.dot(x_ref[...], w_ref[...],
                     preferred_element_type=jnp.float32)
        y_ref[...] = xw * dinv
        dinv_ref[...] = jnp.broadcast_to(dinv, (BR, H))

    return pl.pallas_call(
        body,
        grid=(N // BR,),
        in_specs=[
            pl.BlockSpec((BR, D), lambda i: (i, 0)),
            pl.BlockSpec((D, H), lambda i: (0, 0)),
            pl.BlockSpec((BR, 1), lambda i: (i, 0)),
            pl.BlockSpec((BR, 1), lambda i: (i, 0)),
        ],
        out_specs=[
            pl.BlockSpec((BR, H), lambda i: (i, 0)),
            pl.BlockSpec((BR, H), lambda i: (i, 0)),
        ],
        out_shape=[
            jax.ShapeDtypeStruct((N, H), jnp.float32),
            jax.ShapeDtypeStruct((N, H), jnp.float32),
        ],
    )(x, W1, degp0, degp1)


def _tc_layer2(y1, aggp, dinvb, b1r, W2):
    def body(y_ref, p_ref, dinv_ref, b_ref, w_ref, y2_ref):
        dinv = dinv_ref[...]
        agg = p_ref[...] + y_ref[...]
        h = jnp.maximum(dinv * agg + b_ref[...], 0.0)
        hw = jnp.dot(h, w_ref[...], preferred_element_type=jnp.float32)
        y2 = hw * dinv[:, :C]
        y2_ref[...] = jnp.concatenate([y2, y2], axis=1)

    return pl.pallas_call(
        body,
        grid=(N // BR,),
        in_specs=[
            pl.BlockSpec((BR, H), lambda i: (i, 0)),
            pl.BlockSpec((BR, H), lambda i: (i, 0)),
            pl.BlockSpec((BR, H), lambda i: (i, 0)),
            pl.BlockSpec((1, H), lambda i: (0, 0)),
            pl.BlockSpec((H, C), lambda i: (0, 0)),
        ],
        out_specs=pl.BlockSpec((BR, H), lambda i: (i, 0)),
        out_shape=jax.ShapeDtypeStruct((N, H), jnp.float32),
    )(y1, aggp, dinvb, b1r, W2)


def _tc_layer3(y2w, aggp, dinvb, b2r):
    def body(y_ref, p_ref, dinv_ref, b_ref, o_ref):
        p = p_ref[...]
        agg = p[:, :C] + p[:, C:] + y_ref[...][:, :C]
        o = dinv_ref[...][:, :C] * agg + b_ref[...]
        m = jnp.max(o, axis=1, keepdims=True)
        lse = jnp.log(jnp.sum(jnp.exp(o - m), axis=1, keepdims=True)) + m
        o_ref[...] = o - lse

    return pl.pallas_call(
        body,
        grid=(N // BR,),
        in_specs=[
            pl.BlockSpec((BR, H), lambda i: (i, 0)),
            pl.BlockSpec((BR, H), lambda i: (i, 0)),
            pl.BlockSpec((BR, H), lambda i: (i, 0)),
            pl.BlockSpec((1, C), lambda i: (0, 0)),
        ],
        out_specs=pl.BlockSpec((BR, C), lambda i: (i, 0)),
        out_shape=jax.ShapeDtypeStruct((N, C), jnp.float32),
    )(y2w, aggp, dinvb, b2r)


def kernel(x, edge_index, W1, b1, W2, b2):
    pad_n = E_PAD - E
    pad_iota = np.arange(pad_n, dtype=np.int32)
    dst_deg3d = edge_index[1].reshape(NW, CPW, CHUNK)
    src3d = jnp.concatenate(
        [edge_index[0], jnp.asarray(pad_iota % N)]).reshape(NS, CPWS, ACH)
    dst3d = jnp.concatenate(
        [edge_index[1],
         jnp.asarray(N + pad_iota % N_PAD_ROWS)]).reshape(NS, CPWS, ACH)

    degp = _sc_degree(dst_deg3d)
    degp0 = degp[0, :N].reshape(N, 1)
    degp1 = degp[1, :N].reshape(N, 1)

    y1, dinvb = _tc_layer1(x, W1, degp0, degp1)
    aggp1 = _sc_edge_agg(y1.reshape(2 * N, H // 2), src3d, dst3d)
    y2w = _tc_layer2(y1, aggp1, dinvb, b1.reshape(1, H), W2)
    aggp2 = _sc_edge_agg_es(y2w.reshape(2 * N, C),
                            src3d.reshape(NW, CPW2, ACH),
                            dst3d.reshape(NW, CPW2, ACH))
    return _tc_layer3(y2w, aggp2, dinvb, b2.reshape(1, C))
